# Initial kernel scaffold; baseline (speedup 1.0000x reference)
import jax
import jax.numpy as jnp
from jax.experimental import pallas as pl


def kernel(x, edge_index, W_edge, b_edge, W_node, b_node, W_cat1, b_cat1, W_cat2, b_cat2, W_final, b_final):
    raise NotImplementedError("write your pallas kernel here")


# speedup vs baseline: 5.9459x; 5.9459x over previous
import functools

import jax
import jax.numpy as jnp
from jax import lax
from jax.experimental import pallas as pl
from jax.experimental.pallas import tpu as pltpu
from jax.experimental.pallas import tpu_sc as plsc

N_NODES = 10000
DIM = 128
HALF = DIM // 2
NC = 2
NS = 16
CHUNK = 128
CPT = 157
E_PAD = NS * CPT * CHUNK
ACC_ROWS = 10240
BLK = 1024


def _sc_scatter_body(src_hbm, dst_hbm, w_hbm, out_hbm,
                     src_v, dst_v, rows_v, zero_v, acc, sem):
    c = lax.axis_index("c")
    s = lax.axis_index("s")

    pltpu.sync_copy(src_hbm.at[s], src_v)
    pltpu.sync_copy(dst_hbm.at[s], dst_v)

    def _zrow(i, _):
        def _zcol(j, _):
            zero_v[i, pl.ds(j * 16, 16)] = jnp.zeros((16,), jnp.float32)
            return 0
        return lax.fori_loop(0, HALF // 16, _zcol, 0)
    lax.fori_loop(0, CHUNK, _zrow, 0)

    rows_per_tile = ACC_ROWS // NS
    r0 = s * rows_per_tile
    for b in range(rows_per_tile // CHUNK):
        pltpu.sync_copy(zero_v, acc.at[pl.ds(r0 + b * CHUNK, CHUNK)])
    plsc.subcore_barrier()

    def _body(j, _):
        pltpu.async_copy(w_hbm.at[c].at[src_v.at[j]], rows_v, sem).wait()
        pltpu.sync_copy(rows_v, acc.at[dst_v.at[j]], add=True)
        return 0
    lax.fori_loop(0, CPT, _body, 0)
    plsc.subcore_barrier()

    for b in range(rows_per_tile // CHUNK):
        pltpu.sync_copy(acc.at[pl.ds(r0 + b * CHUNK, CHUNK)],
                        out_hbm.at[c].at[pl.ds(r0 + b * CHUNK, CHUNK)])


_sc_scatter = functools.partial(
    pl.kernel,
    out_type=jax.ShapeDtypeStruct((NC, ACC_ROWS, HALF), jnp.float32),
    mesh=plsc.VectorSubcoreMesh(core_axis_name="c", subcore_axis_name="s"),
    scratch_types=[
        pltpu.VMEM((CPT, CHUNK), jnp.int32),
        pltpu.VMEM((CPT, CHUNK), jnp.int32),
        pltpu.VMEM((CHUNK, HALF), jnp.float32),
        pltpu.VMEM((CHUNK, HALF), jnp.float32),
        pltpu.VMEM_SHARED((ACC_ROWS, HALF), jnp.float32),
        pltpu.SemaphoreType.DMA,
    ],
    compiler_params=pltpu.CompilerParams(use_tc_tiling_on_sc=False),
)(_sc_scatter_body)


def _dense_body(p0, p1, xr, be, wc1, bc1, wn, bn, wc2, bc2, wf, bf, yr):
    f32 = jnp.float32
    S = jnp.concatenate([p0[0], p1[0]], axis=-1) + be[...]
    out = S + jnp.dot(S, wc1[...], preferred_element_type=f32) + bc1[...]
    xn = jnp.dot(xr[...], wn[...], preferred_element_type=f32) + bn[...]
    out = out + xn + jnp.dot(xn, wc2[...], preferred_element_type=f32) + bc2[...]
    yr[...] = jnp.dot(jnp.maximum(out, 0.0), wf[...], preferred_element_type=f32) + bf[...]


def _dense_call(partials, x_pad, b_edge, W_cat1, b_cat1, W_node, b_node,
                W_cat2, b_cat2, W_final, b_final):
    grid = ACC_ROWS // BLK
    row_spec = pl.BlockSpec((BLK, DIM), lambda i: (i, 0))
    w_spec = pl.BlockSpec((DIM, DIM), lambda i: (0, 0))
    b_spec = pl.BlockSpec((1, DIM), lambda i: (0, 0))
    return pl.pallas_call(
        _dense_body,
        grid=(grid,),
        in_specs=[
            pl.BlockSpec((1, BLK, HALF), lambda i: (0, i, 0)),
            pl.BlockSpec((1, BLK, HALF), lambda i: (1, i, 0)),
            row_spec, b_spec, w_spec, b_spec, w_spec, b_spec, w_spec, b_spec,
            w_spec, b_spec,
        ],
        out_specs=row_spec,
        out_shape=jax.ShapeDtypeStruct((ACC_ROWS, DIM), jnp.float32),
    )(partials, partials, x_pad, b_edge, W_cat1, b_cat1, W_node, b_node,
      W_cat2, b_cat2, W_final, b_final)


def kernel(x, edge_index, W_edge, b_edge, W_node, b_node, W_cat1, b_cat1,
           W_cat2, b_cat2, W_final, b_final):
    n, d = W_edge.shape
    e = edge_index.shape[1]
    pad = E_PAD - e
    src = jnp.concatenate([edge_index[0], jnp.zeros((pad,), edge_index.dtype)])
    dst = jnp.concatenate([edge_index[1],
                           jnp.full((pad,), N_NODES, edge_index.dtype)])
    src = src.reshape(NS, CPT, CHUNK)
    dst = dst.reshape(NS, CPT, CHUNK)
    w_halves = W_edge.reshape(n, NC, HALF).transpose(1, 0, 2)

    partials = _sc_scatter(src, dst, w_halves)

    x_pad = jnp.pad(x, ((0, ACC_ROWS - n), (0, 0)))
    y = _dense_call(partials, x_pad,
                    b_edge.reshape(1, d), W_cat1, b_cat1.reshape(1, d),
                    W_node, b_node.reshape(1, d), W_cat2, b_cat2.reshape(1, d),
                    W_final, b_final.reshape(1, d))
    return y[:n]
